# baseline (speedup 1.0000x reference)
import dataclasses

import jax
import jax.numpy as jnp
from jax import lax
from jax.experimental import pallas as pl
from jax.experimental.pallas import tpu as pltpu
from jax.experimental.pallas import tpu_sc as plsc

_B, _L, _K, _H = 16, 256, 32, 32
_N = _L * _L
_CR = 8
_NCHUNK = _L // _CR
_SPLIT = 8
_NB_SC = _B - _SPLIT
_TPB = 32 // _NB_SC
_HH = _H // _TPB
_CH = 8192


def _sc_body(dist_hbm, emb_hbm, out_hbm, emb_v, d0, d1, o0, o1,
             sem_e, sem_d0, sem_d1, sem_o0, sem_o1):
    wid = lax.axis_index("s") * 2 + lax.axis_index("c")
    lb = wid // _TPB
    b = _SPLIT + lb
    hbase = (wid % _TPB) * _HH
    pltpu.async_copy(emb_hbm, emb_v, sem_e).wait()
    pltpu.async_copy(dist_hbm.at[b, pl.ds(0, _CR), :], d0, sem_d0)

    dbufs, dsems = (d0, d1), (sem_d0, sem_d1)
    obufs, osems = (o0, o1), (sem_o0, sem_o1)
    hbases = [jnp.broadcast_to((hbase + h) * _K, (16,)) for h in range(_HH)]

    @pl.loop(0, _NCHUNK // 2)
    def _pair(cc):
        for p in (0, 1):
            c0 = 2 * cc + p
            dbuf, dsem = dbufs[p], dsems[p]
            obuf, osem = obufs[p], osems[p]
            pltpu.make_async_copy(
                dist_hbm.at[b, pl.ds(0, _CR), :], dbuf, dsem).wait()

            @pl.when(c0 + 1 < _NCHUNK)
            def _():
                pltpu.async_copy(
                    dist_hbm.at[b, pl.ds((c0 + 1) * _CR, _CR), :],
                    dbufs[1 - p], dsems[1 - p])

            @pl.when(c0 >= 2)
            def _():
                pltpu.make_async_copy(
                    obuf,
                    out_hbm.at[lb, pl.ds(hbase, _HH), pl.ds(0, _CR), :],
                    osem).wait()

            @pl.loop(0, _CR)
            def _row(r):
                @plsc.parallel_loop(0, _L, 16, unroll=1)
                def _vec(i):
                    dv = dbuf[r, pl.ds(i, 16)]
                    vals = [plsc.load_gather(emb_v, [dv + hbases[h]])
                            for h in range(_HH)]
                    for h in range(_HH):
                        obuf[h, r, pl.ds(i, 16)] = vals[h]

            pltpu.async_copy(
                obuf,
                out_hbm.at[lb, pl.ds(hbase, _HH), pl.ds(c0 * _CR, _CR), :],
                osem)

    for p in (0, 1):
        pltpu.make_async_copy(
            obufs[p],
            out_hbm.at[lb, pl.ds(hbase, _HH), pl.ds(0, _CR), :],
            osems[p]).wait()


def _tc_body(dist_ref, embT_ref, out_ref):
    d = dist_ref[0]
    iota = jax.lax.broadcasted_iota(jnp.int32, (_K, _CH), 0)
    oh = (iota == d).astype(jnp.float32)
    out_ref[0] = jnp.dot(embT_ref[...], oh, preferred_element_type=jnp.float32)


def kernel(hop_dist, hop_emb):
    B, L, _ = hop_dist.shape
    K, H = hop_emb.shape
    embT = hop_emb.T
    embT_flat = embT.reshape(-1)

    mesh = plsc.VectorSubcoreMesh(core_axis_name="c", subcore_axis_name="s")
    cp = pltpu.CompilerParams()
    if "needs_layout_passes" in pltpu.CompilerParams.__dataclass_fields__:
        cp = dataclasses.replace(cp, needs_layout_passes=False)
    sc = pl.kernel(
        _sc_body,
        out_type=jax.ShapeDtypeStruct((_NB_SC, H, L, L), jnp.float32),
        mesh=mesh,
        compiler_params=cp,
        scratch_types=[
            pltpu.VMEM((H * K,), jnp.float32),
            pltpu.VMEM((_CR, _L), jnp.int32),
            pltpu.VMEM((_CR, _L), jnp.int32),
            pltpu.VMEM((_HH, _CR, _L), jnp.float32),
            pltpu.VMEM((_HH, _CR, _L), jnp.float32),
            pltpu.SemaphoreType.DMA,
            pltpu.SemaphoreType.DMA,
            pltpu.SemaphoreType.DMA,
            pltpu.SemaphoreType.DMA,
            pltpu.SemaphoreType.DMA,
        ],
    )
    sc_out = sc(hop_dist, embT_flat)

    dist_flat = hop_dist[:_SPLIT].reshape(_SPLIT, 1, _N)
    tc_out = pl.pallas_call(
        _tc_body,
        grid=(_SPLIT, _N // _CH),
        in_specs=[
            pl.BlockSpec((1, 1, _CH), lambda b, c: (b, 0, c)),
            pl.BlockSpec((H, K), lambda b, c: (0, 0)),
        ],
        out_specs=pl.BlockSpec((1, H, _CH), lambda b, c: (b, 0, c)),
        out_shape=jax.ShapeDtypeStruct((_SPLIT, H, _N), jnp.float32),
    )(dist_flat, embT).reshape(_SPLIT, H, L, L)

    return jnp.concatenate([tc_out, sc_out], axis=0)


# speedup vs baseline: 3.1262x; 3.1262x over previous
import dataclasses

import jax
import jax.numpy as jnp
from jax import lax
from jax.experimental import pallas as pl
from jax.experimental.pallas import tpu as pltpu
from jax.experimental.pallas import tpu_sc as plsc

_B, _L, _K, _H = 16, 256, 32, 32
_CR = 8
_NCHUNK = _L // _CR
_HHALF = _H // 2


def _sc_body(dist_hbm, emb_hbm, out_hbm, emb_v, d0, d1, o0, o1,
             sem_e, sem_d0, sem_d1, sem_o0, sem_o1):
    wid = lax.axis_index("s") * 2 + lax.axis_index("c")
    b = wid // 2
    hbase = (wid % 2) * _HHALF
    pltpu.async_copy(emb_hbm, emb_v, sem_e).wait()
    pltpu.async_copy(dist_hbm.at[b, pl.ds(0, _CR), :], d0, sem_d0)

    dbufs, dsems = (d0, d1), (sem_d0, sem_d1)
    obufs, osems = (o0, o1), (sem_o0, sem_o1)
    hbases = [jnp.broadcast_to((hbase + h) * _K, (16,)) for h in range(_HHALF)]

    @pl.loop(0, _NCHUNK // 2)
    def _pair(cc):
        for p in (0, 1):
            c0 = 2 * cc + p
            dbuf, dsem = dbufs[p], dsems[p]
            obuf, osem = obufs[p], osems[p]
            pltpu.make_async_copy(
                dist_hbm.at[b, pl.ds(0, _CR), :], dbuf, dsem).wait()

            @pl.when(c0 + 1 < _NCHUNK)
            def _():
                pltpu.async_copy(
                    dist_hbm.at[b, pl.ds((c0 + 1) * _CR, _CR), :],
                    dbufs[1 - p], dsems[1 - p])

            @pl.when(c0 >= 2)
            def _():
                pltpu.make_async_copy(
                    obuf,
                    out_hbm.at[b, pl.ds(hbase, _HHALF), pl.ds(0, _CR), :],
                    osem).wait()

            @pl.loop(0, _CR)
            def _row(r):
                @plsc.parallel_loop(0, _L, 16, unroll=1)
                def _vec(i):
                    dv = dbuf[r, pl.ds(i, 16)]
                    vals = [plsc.load_gather(emb_v, [dv + hbases[h]])
                            for h in range(_HHALF)]
                    for h in range(_HHALF):
                        obuf[h, r, pl.ds(i, 16)] = vals[h]

            pltpu.async_copy(
                obuf,
                out_hbm.at[b, pl.ds(hbase, _HHALF), pl.ds(c0 * _CR, _CR), :],
                osem)

    for p in (0, 1):
        pltpu.make_async_copy(
            obufs[p],
            out_hbm.at[b, pl.ds(hbase, _HHALF), pl.ds(0, _CR), :],
            osems[p]).wait()


def kernel(hop_dist, hop_emb):
    B, L, _ = hop_dist.shape
    K, H = hop_emb.shape
    mesh = plsc.VectorSubcoreMesh(core_axis_name="c", subcore_axis_name="s")
    cp = pltpu.CompilerParams()
    if "needs_layout_passes" in pltpu.CompilerParams.__dataclass_fields__:
        cp = dataclasses.replace(cp, needs_layout_passes=False)
    k = pl.kernel(
        _sc_body,
        out_type=jax.ShapeDtypeStruct((B, H, L, L), jnp.float32),
        mesh=mesh,
        compiler_params=cp,
        scratch_types=[
            pltpu.VMEM((H * K,), jnp.float32),
            pltpu.VMEM((_CR, _L), jnp.int32),
            pltpu.VMEM((_CR, _L), jnp.int32),
            pltpu.VMEM((_HHALF, _CR, _L), jnp.float32),
            pltpu.VMEM((_HHALF, _CR, _L), jnp.float32),
            pltpu.SemaphoreType.DMA,
            pltpu.SemaphoreType.DMA,
            pltpu.SemaphoreType.DMA,
            pltpu.SemaphoreType.DMA,
            pltpu.SemaphoreType.DMA,
        ],
    )
    embT_flat = hop_emb.T.reshape(-1)
    return k(hop_dist, embT_flat)
